# Initial kernel scaffold; baseline (speedup 1.0000x reference)
import jax
import jax.numpy as jnp
from jax.experimental import pallas as pl


def kernel(input):
    raise NotImplementedError("write your pallas kernel here")


# speedup vs baseline: 457.7969x; 457.7969x over previous
import functools

import jax
import jax.numpy as jnp
from jax import lax
from jax.experimental import pallas as pl
from jax.experimental.pallas import tpu as pltpu
from jax.experimental.pallas import tpu_sc as plsc

_NUM_CLASSES = 256
_NUM_REGIONS = 512
_N = 16777216

_NC = 2
_NS = 16
_L = 16
_NW = _NC * _NS
_PER_W = _N // _NW
_CHUNK = 16384
_NCHUNK = _PER_W // _CHUNK
_VECS = _CHUNK // _L
_UNROLL = 4

_ROWS, _COLS = 2048, 8192
_BLK_ROWS = 256


def _minmax_body(x_ref, mn_ref, mx_ref):
    i = pl.program_id(0)
    bmn = jnp.min(x_ref[...])
    bmx = jnp.max(x_ref[...])

    @pl.when(i == 0)
    def _init():
        mn_ref[0, 0] = bmn
        mx_ref[0, 0] = bmx

    @pl.when(i > 0)
    def _acc():
        mn_ref[0, 0] = jnp.minimum(mn_ref[0, 0], bmn)
        mx_ref[0, 0] = jnp.maximum(mx_ref[0, 0], bmx)


_minmax = pl.pallas_call(
    _minmax_body,
    grid=(_ROWS // _BLK_ROWS,),
    in_specs=[pl.BlockSpec((_BLK_ROWS, _COLS), lambda i: (i, 0))],
    out_specs=[pl.BlockSpec(memory_space=pltpu.SMEM)] * 2,
    out_shape=[jax.ShapeDtypeStruct((1, 1), jnp.float32)] * 2,
)

_mesh = plsc.VectorSubcoreMesh(core_axis_name="c", subcore_axis_name="s")


@functools.partial(
    pl.kernel,
    mesh=_mesh,
    out_type=jax.ShapeDtypeStruct((_N,), jnp.int32),
    compiler_params=pltpu.CompilerParams(needs_layout_passes=False),
    scratch_types=[
        pltpu.VMEM((_NUM_REGIONS,), jnp.float32),
        pltpu.VMEM((_NUM_REGIONS,), jnp.int32),
        pltpu.VMEM((_CHUNK,), jnp.float32),
        pltpu.VMEM((_CHUNK,), jnp.int32),
    ],
)
def _sc_assign(x_hbm, thr_hbm, cls_hbm, out_hbm, thr_v, cls_v, xbuf, obuf):
    wid = lax.axis_index("s") * _NC + lax.axis_index("c")
    base = wid * _PER_W
    pltpu.sync_copy(thr_hbm, thr_v)
    pltpu.sync_copy(cls_hbm, cls_v)

    def chunk_body(g, carry):
        off = base + g * _CHUNK
        pltpu.sync_copy(x_hbm.at[pl.ds(off, _CHUNK)], xbuf)

        def vec_body(j, carry2):
            s0 = j * (_L * _UNROLL)
            for ui in range(_UNROLL):
                s = s0 + ui * _L
                x = xbuf[pl.ds(s, _L)]
                pos = jnp.zeros((_L,), jnp.int32)
                for kbit in range(8, -1, -1):
                    probe = pos | ((1 << kbit) - 1)
                    t = plsc.load_gather(thr_v, [probe])
                    pos = jnp.where(t < x, pos | (1 << kbit), pos)
                obuf[pl.ds(s, _L)] = plsc.load_gather(cls_v, [pos])
            return carry2

        lax.fori_loop(0, _VECS // _UNROLL, vec_body, 0)
        pltpu.sync_copy(obuf, out_hbm.at[pl.ds(off, _CHUNK)])
        return carry

    lax.fori_loop(0, _NCHUNK, chunk_body, 0)


def kernel(input):
    mn, mx = _minmax(input.reshape(_ROWS, _COLS))
    dmn = mn[0, 0]
    dmx = mx[0, 0]
    k = jax.random.key(1)
    k1, k2 = jax.random.split(k)
    u_sorted = jnp.sort(jax.random.uniform(k1, (_NUM_REGIONS - 1,), dtype=jnp.float32))
    cls = jax.random.randint(k2, (_NUM_REGIONS,), 0, _NUM_CLASSES, dtype=jnp.int32)
    thr = u_sorted * (dmx - dmn) + dmn
    thr_pad = jnp.concatenate([thr, jnp.full((1,), jnp.inf, dtype=jnp.float32)])
    return _sc_assign(input, thr_pad, cls)


# speedup vs baseline: 1203.6682x; 2.6293x over previous
import functools

import jax
import jax.numpy as jnp
from jax import lax
from jax.experimental import pallas as pl
from jax.experimental.pallas import tpu as pltpu
from jax.experimental.pallas import tpu_sc as plsc

_NUM_CLASSES = 256
_NUM_REGIONS = 512
_N = 16777216

_NC = 2
_NS = 16
_L = 16
_NW = _NC * _NS
_PER_W = _N // _NW
_CHUNK = 16384
_NCHUNK = _PER_W // _CHUNK
_VECS = _CHUNK // _L
_UNROLL = 4

_ROWS, _COLS = 2048, 8192
_BLK_ROWS = 256


def _minmax_body(x_ref, mn_ref, mx_ref):
    i = pl.program_id(0)
    bmn = jnp.min(x_ref[...])
    bmx = jnp.max(x_ref[...])

    @pl.when(i == 0)
    def _init():
        mn_ref[0, 0] = bmn
        mx_ref[0, 0] = bmx

    @pl.when(i > 0)
    def _acc():
        mn_ref[0, 0] = jnp.minimum(mn_ref[0, 0], bmn)
        mx_ref[0, 0] = jnp.maximum(mx_ref[0, 0], bmx)


_minmax = pl.pallas_call(
    _minmax_body,
    grid=(_ROWS // _BLK_ROWS,),
    in_specs=[pl.BlockSpec((_BLK_ROWS, _COLS), lambda i: (i, 0))],
    out_specs=[pl.BlockSpec(memory_space=pltpu.SMEM)] * 2,
    out_shape=[jax.ShapeDtypeStruct((1, 1), jnp.float32)] * 2,
)

_mesh = plsc.VectorSubcoreMesh(core_axis_name="c", subcore_axis_name="s")


@functools.partial(
    pl.kernel,
    mesh=_mesh,
    out_type=jax.ShapeDtypeStruct((_N,), jnp.int32),
    compiler_params=pltpu.CompilerParams(needs_layout_passes=False),
    scratch_types=[
        pltpu.VMEM((_NUM_REGIONS,), jnp.float32),
        pltpu.VMEM((_NUM_REGIONS,), jnp.int32),
        pltpu.VMEM((_CHUNK,), jnp.float32),
        pltpu.VMEM((_CHUNK,), jnp.int32),
    ],
)
def _sc_assign(x_hbm, thr_hbm, cls_hbm, out_hbm, thr_v, cls_v, xbuf, obuf):
    wid = lax.axis_index("s") * _NC + lax.axis_index("c")
    base = wid * _PER_W
    pltpu.sync_copy(thr_hbm, thr_v)
    pltpu.sync_copy(cls_hbm, cls_v)

    def chunk_body(g, carry):
        off = base + g * _CHUNK
        pltpu.sync_copy(x_hbm.at[pl.ds(off, _CHUNK)], xbuf)

        @plsc.parallel_loop(0, _CHUNK, step=_L, unroll=_UNROLL)
        def _vec(s):
            x = xbuf[pl.ds(s, _L)]
            pos = jnp.zeros((_L,), jnp.int32)
            for kbit in range(8, -1, -1):
                probe = pos | ((1 << kbit) - 1)
                t = plsc.load_gather(thr_v, [probe])
                pos = jnp.where(t < x, pos | (1 << kbit), pos)
            obuf[pl.ds(s, _L)] = plsc.load_gather(cls_v, [pos])

        pltpu.sync_copy(obuf, out_hbm.at[pl.ds(off, _CHUNK)])
        return carry

    lax.fori_loop(0, _NCHUNK, chunk_body, 0)


def kernel(input):
    mn, mx = _minmax(input.reshape(_ROWS, _COLS))
    dmn = mn[0, 0]
    dmx = mx[0, 0]
    k = jax.random.key(1)
    k1, k2 = jax.random.split(k)
    u_sorted = jnp.sort(jax.random.uniform(k1, (_NUM_REGIONS - 1,), dtype=jnp.float32))
    cls = jax.random.randint(k2, (_NUM_REGIONS,), 0, _NUM_CLASSES, dtype=jnp.int32)
    thr = u_sorted * (dmx - dmn) + dmn
    thr_pad = jnp.concatenate([thr, jnp.full((1,), jnp.inf, dtype=jnp.float32)])
    return _sc_assign(input, thr_pad, cls)
